# initial kernel scaffold (unmeasured)
import jax
import jax.numpy as jnp
from jax import lax
from jax.experimental import pallas as pl
from jax.experimental.pallas import tpu as pltpu

M = 1024
H = M // 2
Q = M // 4


def kernel(x, w_mat):
    m, k = x.shape
    _, n = w_mat.shape
    assert m == M

    def body(x_ref, w_ref, out_ref,
             h_send, h_recv, q_send, q_recv, g_half, g_qrecv, g_recv,
             send_sems, recv_sems):
        d = lax.axis_index("i")
        p1 = d ^ 1
        p2 = 3 - d
        bit = d >> 1
        par = (d & 1) ^ bit
        keep_off = par * H
        other_off = (1 - par) * H
        q_within = bit * Q
        q_other_within = (1 - bit) * Q

        def exchange(src, dst, partner, s):
            rdma = pltpu.make_async_remote_copy(
                src_ref=src, dst_ref=dst,
                send_sem=send_sems.at[s], recv_sem=recv_sems.at[s],
                device_id=(partner,), device_id_type=pl.DeviceIdType.MESH,
            )
            rdma.start()
            return rdma

        p_other = jnp.dot(x_ref[pl.ds(other_off, H), :], w_ref[...],
                          preferred_element_type=jnp.float32)
        h_send[...] = p_other.astype(jnp.bfloat16)
        r1 = exchange(h_send, h_recv, p1, 0)
        p_keep = jnp.dot(x_ref[pl.ds(keep_off, H), :], w_ref[...],
                         preferred_element_type=jnp.float32)
        r1.wait()
        acc_half = p_keep + h_recv[...].astype(jnp.float32)

        q_send[...] = lax.dynamic_slice_in_dim(
            acc_half, q_other_within, Q, 0).astype(jnp.bfloat16)
        r2 = exchange(q_send, q_recv, p2, 1)
        r2.wait()
        q_mine = (lax.dynamic_slice_in_dim(acc_half, q_within, Q, 0)
                  + q_recv[...].astype(jnp.float32))
        r = jnp.maximum(q_mine, 0.0)

        out_ref[pl.ds(keep_off + q_within, Q), :] = r
        g_half[pl.ds(q_within, Q), :] = r.astype(jnp.bfloat16)

        r3 = exchange(g_half.at[pl.ds(q_within, Q), :], g_qrecv, p2, 2)
        r3.wait()
        out_ref[pl.ds(keep_off + q_other_within, Q), :] = (
            g_qrecv[...].astype(jnp.float32))
        g_half[pl.ds(q_other_within, Q), :] = g_qrecv[...]

        r4 = exchange(g_half, g_recv, p1, 3)
        r4.wait()
        out_ref[pl.ds(other_off, H), :] = g_recv[...].astype(jnp.float32)

    return pl.pallas_call(
        body,
        out_shape=jax.ShapeDtypeStruct((M, n), jnp.float32),
        in_specs=[
            pl.BlockSpec(memory_space=pltpu.VMEM),
            pl.BlockSpec(memory_space=pltpu.VMEM),
        ],
        out_specs=pl.BlockSpec(memory_space=pltpu.VMEM),
        scratch_shapes=[
            pltpu.VMEM((H, n), jnp.bfloat16),
            pltpu.VMEM((H, n), jnp.bfloat16),
            pltpu.VMEM((Q, n), jnp.bfloat16),
            pltpu.VMEM((Q, n), jnp.bfloat16),
            pltpu.VMEM((H, n), jnp.bfloat16),
            pltpu.VMEM((Q, n), jnp.bfloat16),
            pltpu.VMEM((H, n), jnp.bfloat16),
            pltpu.SemaphoreType.DMA((4,)),
            pltpu.SemaphoreType.DMA((4,)),
        ],
    )(x, w_mat)


# baseline (device time: 51309 ns/iter reference)
import jax
import jax.numpy as jnp
from jax import lax
from jax.experimental import pallas as pl
from jax.experimental.pallas import tpu as pltpu

M = 1024
H = M // 2
Q = M // 4


def kernel(x, w_mat):
    m, k = x.shape
    _, n = w_mat.shape
    assert m == M

    def body(x_ref, w_ref, out_ref,
             h_send, h_recv, q_send, q_recv, g_half, g_qrecv, g_recv,
             acc_ref, send_sems, recv_sems):
        d = lax.axis_index("i")
        p1 = d ^ 1
        p2 = 3 - d
        bit = d >> 1
        par = (d & 1) ^ bit
        keep_off = par * H
        other_off = (1 - par) * H
        q_within = bit * Q
        q_other_within = (1 - bit) * Q

        def exchange(src, dst, partner, s):
            rdma = pltpu.make_async_remote_copy(
                src_ref=src, dst_ref=dst,
                send_sem=send_sems.at[s], recv_sem=recv_sems.at[s],
                device_id=(partner,), device_id_type=pl.DeviceIdType.MESH,
            )
            rdma.start()
            return rdma

        p_other = jnp.dot(x_ref[pl.ds(other_off, H), :], w_ref[...],
                          preferred_element_type=jnp.float32)
        h_send[...] = p_other.astype(jnp.bfloat16)
        r1 = exchange(h_send, h_recv, p1, 0)
        p_keep = jnp.dot(x_ref[pl.ds(keep_off, H), :], w_ref[...],
                         preferred_element_type=jnp.float32)
        r1.wait()
        acc_ref[...] = p_keep + h_recv[...].astype(jnp.float32)

        q_send[...] = acc_ref[pl.ds(q_other_within, Q), :].astype(jnp.bfloat16)
        r2 = exchange(q_send, q_recv, p2, 1)
        r2.wait()
        q_mine = (acc_ref[pl.ds(q_within, Q), :]
                  + q_recv[...].astype(jnp.float32))
        r = jnp.maximum(q_mine, 0.0)

        out_ref[pl.ds(keep_off + q_within, Q), :] = r
        g_half[pl.ds(q_within, Q), :] = r.astype(jnp.bfloat16)

        r3 = exchange(g_half.at[pl.ds(q_within, Q), :], g_qrecv, p2, 2)
        r3.wait()
        out_ref[pl.ds(keep_off + q_other_within, Q), :] = (
            g_qrecv[...].astype(jnp.float32))
        g_half[pl.ds(q_other_within, Q), :] = g_qrecv[...]

        r4 = exchange(g_half, g_recv, p1, 3)
        r4.wait()
        out_ref[pl.ds(other_off, H), :] = g_recv[...].astype(jnp.float32)

    return pl.pallas_call(
        body,
        out_shape=jax.ShapeDtypeStruct((M, n), jnp.float32),
        in_specs=[
            pl.BlockSpec(memory_space=pltpu.VMEM),
            pl.BlockSpec(memory_space=pltpu.VMEM),
        ],
        out_specs=pl.BlockSpec(memory_space=pltpu.VMEM),
        scratch_shapes=[
            pltpu.VMEM((H, n), jnp.bfloat16),
            pltpu.VMEM((H, n), jnp.bfloat16),
            pltpu.VMEM((Q, n), jnp.bfloat16),
            pltpu.VMEM((Q, n), jnp.bfloat16),
            pltpu.VMEM((H, n), jnp.bfloat16),
            pltpu.VMEM((Q, n), jnp.bfloat16),
            pltpu.VMEM((H, n), jnp.bfloat16),
            pltpu.VMEM((H, n), jnp.float32),
            pltpu.SemaphoreType.DMA((4,)),
            pltpu.SemaphoreType.DMA((4,)),
        ],
    )(x, w_mat)


# device time: 31911 ns/iter; 1.6079x vs baseline; 1.6079x over previous
import jax
import jax.numpy as jnp
from jax import lax
from jax.experimental import pallas as pl
from jax.experimental.pallas import tpu as pltpu

M = 1024
H = M // 2
Q = M // 4


def kernel(x, w_mat):
    m, k = x.shape
    _, n = w_mat.shape
    assert m == M
    n2 = n // 2

    def body(x_ref, w_ref, out_ref,
             ha_send, ha_recv, qa_send, qa_recv, ga_half, ga_qrecv, ga_recv,
             acc_a,
             hb_send, hb_recv, qb_send, qb_recv, gb_half, gb_qrecv, gb_recv,
             acc_b,
             send_sems, recv_sems):
        d = lax.axis_index("i")
        p1 = d ^ 1
        p2 = 3 - d
        bit = d >> 1
        low = d & 1

        barrier_sem = pltpu.get_barrier_semaphore()
        for nbr in (p1, p2):
            pl.semaphore_signal(
                barrier_sem, inc=1,
                device_id=(nbr,), device_id_type=pl.DeviceIdType.MESH,
            )
        pl.semaphore_wait(barrier_sem, 2)

        cfgs = [
            dict(par=low ^ bit, qw=bit * Q, s1=p1, s2=p2,
                 cols=slice(0, n2), sb=0,
                 bufs=(ha_send, ha_recv, qa_send, qa_recv,
                       ga_half, ga_qrecv, ga_recv, acc_a)),
            dict(par=bit, qw=low * Q, s1=p2, s2=p1,
                 cols=slice(n2, n), sb=4,
                 bufs=(hb_send, hb_recv, qb_send, qb_recv,
                       gb_half, gb_qrecv, gb_recv, acc_b)),
        ]
        for c in cfgs:
            c["keep"] = c["par"] * H
            c["other"] = (1 - c["par"]) * H
            c["qo"] = Q - c["qw"]

        def exchange(src, dst, partner, s):
            rdma = pltpu.make_async_remote_copy(
                src_ref=src, dst_ref=dst,
                send_sem=send_sems.at[s], recv_sem=recv_sems.at[s],
                device_id=(partner,), device_id_type=pl.DeviceIdType.MESH,
            )
            rdma.start()
            return rdma

        pend = []

        r1 = []
        for c in cfgs:
            (h_send, _, _, _, _, _, _, _) = c["bufs"]
            p_other = jnp.dot(x_ref[pl.ds(c["other"], H), :],
                              w_ref[:, c["cols"]],
                              preferred_element_type=jnp.float32)
            h_send[...] = p_other.astype(jnp.bfloat16)
            r1.append(exchange(h_send, c["bufs"][1], c["s1"], c["sb"] + 0))
        keeps = [
            jnp.dot(x_ref[pl.ds(c["keep"], H), :], w_ref[:, c["cols"]],
                    preferred_element_type=jnp.float32)
            for c in cfgs
        ]
        for c, r, p_keep in zip(cfgs, r1, keeps):
            (_, h_recv, _, _, _, _, _, acc) = c["bufs"]
            r.wait_recv()
            acc[...] = p_keep + h_recv[...].astype(jnp.float32)
        pend += r1

        r2 = []
        for c in cfgs:
            (_, _, q_send, _, _, _, _, acc) = c["bufs"]
            q_send[...] = acc[pl.ds(c["qo"], Q), :].astype(jnp.bfloat16)
            r2.append(exchange(q_send, c["bufs"][3], c["s2"], c["sb"] + 1))
        for c, r in zip(cfgs, r2):
            (_, _, _, q_recv, g_half, _, _, acc) = c["bufs"]
            r.wait_recv()
            red = acc[pl.ds(c["qw"], Q), :] + q_recv[...].astype(jnp.float32)
            red = jnp.maximum(red, 0.0)
            out_ref[pl.ds(c["keep"] + c["qw"], Q), c["cols"]] = red
            g_half[pl.ds(c["qw"], Q), :] = red.astype(jnp.bfloat16)
        pend += r2

        r3 = [
            exchange(c["bufs"][4].at[pl.ds(c["qw"], Q), :],
                     c["bufs"][5], c["s2"], c["sb"] + 2)
            for c in cfgs
        ]
        for c, r in zip(cfgs, r3):
            (_, _, _, _, g_half, g_qrecv, _, _) = c["bufs"]
            r.wait_recv()
            out_ref[pl.ds(c["keep"] + c["qo"], Q), c["cols"]] = (
                g_qrecv[...].astype(jnp.float32))
            g_half[pl.ds(c["qo"], Q), :] = g_qrecv[...]
        pend += r3

        r4 = [
            exchange(c["bufs"][4], c["bufs"][6], c["s1"], c["sb"] + 3)
            for c in cfgs
        ]
        for c, r in zip(cfgs, r4):
            r.wait_recv()
            out_ref[pl.ds(c["other"], H), c["cols"]] = (
                c["bufs"][6][...].astype(jnp.float32))
        pend += r4

        for r in pend:
            r.wait_send()

    half_bufs = [
        pltpu.VMEM((H, n2), jnp.bfloat16),
        pltpu.VMEM((H, n2), jnp.bfloat16),
        pltpu.VMEM((Q, n2), jnp.bfloat16),
        pltpu.VMEM((Q, n2), jnp.bfloat16),
        pltpu.VMEM((H, n2), jnp.bfloat16),
        pltpu.VMEM((Q, n2), jnp.bfloat16),
        pltpu.VMEM((H, n2), jnp.bfloat16),
        pltpu.VMEM((H, n2), jnp.float32),
    ]
    return pl.pallas_call(
        body,
        out_shape=jax.ShapeDtypeStruct((M, n), jnp.float32),
        in_specs=[
            pl.BlockSpec(memory_space=pltpu.VMEM),
            pl.BlockSpec(memory_space=pltpu.VMEM),
        ],
        out_specs=pl.BlockSpec(memory_space=pltpu.VMEM),
        scratch_shapes=half_bufs + half_bufs + [
            pltpu.SemaphoreType.DMA((8,)),
            pltpu.SemaphoreType.DMA((8,)),
        ],
        compiler_params=pltpu.CompilerParams(collective_id=0),
    )(x, w_mat)


# device time: 29444 ns/iter; 1.7426x vs baseline; 1.0838x over previous
import jax
import jax.numpy as jnp
from jax import lax
from jax.experimental import pallas as pl
from jax.experimental.pallas import tpu as pltpu

M = 1024
H = M // 2
Q = M // 4
LANES = 4
BUFS_PER_LANE = 8


def kernel(x, w_mat):
    m, k = x.shape
    _, n = w_mat.shape
    assert m == M
    nc = n // LANES

    def body(x_ref, w_ref, out_ref, *scratch):
        lane_bufs = [
            scratch[i * BUFS_PER_LANE:(i + 1) * BUFS_PER_LANE]
            for i in range(LANES)
        ]
        send_sems, recv_sems = scratch[LANES * BUFS_PER_LANE:]

        d = lax.axis_index("i")
        p1 = d ^ 1
        p2 = 3 - d
        bit = d >> 1
        low = d & 1

        barrier_sem = pltpu.get_barrier_semaphore()
        for nbr in (p1, p2):
            pl.semaphore_signal(
                barrier_sem, inc=1,
                device_id=(nbr,), device_id_type=pl.DeviceIdType.MESH,
            )
        pl.semaphore_wait(barrier_sem, 2)

        cfgs = []
        for lane in range(LANES):
            sched_a = lane < 2
            par = (low ^ bit) if sched_a else bit
            qw = (bit * Q) if sched_a else (low * Q)
            cfgs.append(dict(
                par=par, qw=qw, qo=Q - qw,
                keep=par * H, other=(1 - par) * H,
                s1=p1 if sched_a else p2,
                s2=p2 if sched_a else p1,
                cols=slice(lane * nc, (lane + 1) * nc),
                sb=4 * lane,
                bufs=lane_bufs[lane],
            ))

        def exchange(src, dst, partner, s):
            rdma = pltpu.make_async_remote_copy(
                src_ref=src, dst_ref=dst,
                send_sem=send_sems.at[s], recv_sem=recv_sems.at[s],
                device_id=(partner,), device_id_type=pl.DeviceIdType.MESH,
            )
            rdma.start()
            return rdma

        pend = []

        r1 = []
        for c in cfgs:
            h_send = c["bufs"][0]
            p_other = jnp.dot(x_ref[pl.ds(c["other"], H), :],
                              w_ref[:, c["cols"]],
                              preferred_element_type=jnp.float32)
            h_send[...] = p_other.astype(jnp.bfloat16)
            r1.append(exchange(h_send, c["bufs"][1], c["s1"], c["sb"] + 0))
        pend += r1
        keeps = [
            jnp.dot(x_ref[pl.ds(c["keep"], H), :], w_ref[:, c["cols"]],
                    preferred_element_type=jnp.float32)
            for c in cfgs
        ]

        r2 = []
        for c, r, p_keep in zip(cfgs, r1, keeps):
            (_, h_recv, q_send, _, _, _, _, acc) = c["bufs"]
            r.wait_recv()
            acc[...] = p_keep + h_recv[...].astype(jnp.float32)
            q_send[...] = acc[pl.ds(c["qo"], Q), :].astype(jnp.bfloat16)
            r2.append(exchange(q_send, c["bufs"][3], c["s2"], c["sb"] + 1))
        pend += r2

        r3 = []
        for c, r in zip(cfgs, r2):
            (_, _, _, q_recv, g_half, _, _, acc) = c["bufs"]
            r.wait_recv()
            red = acc[pl.ds(c["qw"], Q), :] + q_recv[...].astype(jnp.float32)
            red = jnp.maximum(red, 0.0)
            out_ref[pl.ds(c["keep"] + c["qw"], Q), c["cols"]] = red
            g_half[pl.ds(c["qw"], Q), :] = red.astype(jnp.bfloat16)
            r3.append(exchange(g_half.at[pl.ds(c["qw"], Q), :],
                               c["bufs"][5], c["s2"], c["sb"] + 2))
        pend += r3

        r4 = []
        for c, r in zip(cfgs, r3):
            (_, _, _, _, g_half, g_qrecv, _, _) = c["bufs"]
            r.wait_recv()
            out_ref[pl.ds(c["keep"] + c["qo"], Q), c["cols"]] = (
                g_qrecv[...].astype(jnp.float32))
            g_half[pl.ds(c["qo"], Q), :] = g_qrecv[...]
            r4.append(exchange(g_half, c["bufs"][6], c["s1"], c["sb"] + 3))
        pend += r4

        for c, r in zip(cfgs, r4):
            r.wait_recv()
            out_ref[pl.ds(c["other"], H), c["cols"]] = (
                c["bufs"][6][...].astype(jnp.float32))

        for r in pend:
            r.wait_send()

    lane_scratch = [
        pltpu.VMEM((H, nc), jnp.bfloat16),
        pltpu.VMEM((H, nc), jnp.bfloat16),
        pltpu.VMEM((Q, nc), jnp.bfloat16),
        pltpu.VMEM((Q, nc), jnp.bfloat16),
        pltpu.VMEM((H, nc), jnp.bfloat16),
        pltpu.VMEM((Q, nc), jnp.bfloat16),
        pltpu.VMEM((H, nc), jnp.bfloat16),
        pltpu.VMEM((H, nc), jnp.float32),
    ]
    return pl.pallas_call(
        body,
        out_shape=jax.ShapeDtypeStruct((M, n), jnp.float32),
        in_specs=[
            pl.BlockSpec(memory_space=pltpu.VMEM),
            pl.BlockSpec(memory_space=pltpu.VMEM),
        ],
        out_specs=pl.BlockSpec(memory_space=pltpu.VMEM),
        scratch_shapes=lane_scratch * LANES + [
            pltpu.SemaphoreType.DMA((4 * LANES,)),
            pltpu.SemaphoreType.DMA((4 * LANES,)),
        ],
        compiler_params=pltpu.CompilerParams(collective_id=0),
    )(x, w_mat)


# device time: 28071 ns/iter; 1.8278x vs baseline; 1.0489x over previous
import jax
import jax.numpy as jnp
from jax import lax
from jax.experimental import pallas as pl
from jax.experimental.pallas import tpu as pltpu

M = 1024
H = M // 2
Q = M // 4
LANES = 4
BUFS_PER_LANE = 5
SLOTS = 5


def kernel(x, w_mat):
    m, k = x.shape
    _, n = w_mat.shape
    assert m == M
    nc = n // LANES

    def body(x_ref, w_ref, out_ref, *scratch):
        lane_bufs = [
            scratch[i * BUFS_PER_LANE:(i + 1) * BUFS_PER_LANE]
            for i in range(LANES)
        ]
        send_sems, recv_sems = scratch[LANES * BUFS_PER_LANE:]

        d = lax.axis_index("i")
        p1 = d ^ 1
        p2 = 3 - d
        diag = p2 ^ 1
        bit = d >> 1
        low = d & 1

        barrier_sem = pltpu.get_barrier_semaphore()
        for nbr in (p1, p2):
            pl.semaphore_signal(
                barrier_sem, inc=1,
                device_id=(nbr,), device_id_type=pl.DeviceIdType.MESH,
            )
        pl.semaphore_wait(barrier_sem, 2)

        def gq_a(e):
            return ((e & 1) ^ (e >> 1)) * H + (e >> 1) * Q

        def gq_b(e):
            return (e >> 1) * H + (e & 1) * Q

        cfgs = []
        for lane in range(LANES):
            a = lane < 2
            par = (low ^ bit) if a else bit
            qw = (bit * Q) if a else (low * Q)
            gq = gq_a if a else gq_b
            cfgs.append(dict(
                par=par, qw=qw, qo=Q - qw,
                keep=par * H, other=(1 - par) * H,
                s1=p1 if a else p2,
                s2=p2 if a else p1,
                dp=p1 if a else p2,
                rp=p2 if a else p1,
                gq=gq,
                cols=slice(lane * nc, (lane + 1) * nc),
                sb=SLOTS * lane,
                bufs=lane_bufs[lane],
            ))

        def rdma(src_rows, dst_rows, c, slot, partner):
            return pltpu.make_async_remote_copy(
                src_ref=out_ref.at[pl.ds(src_rows, Q), c["cols"]],
                dst_ref=out_ref.at[pl.ds(dst_rows, Q), c["cols"]],
                send_sem=send_sems.at[c["sb"] + slot],
                recv_sem=recv_sems.at[c["sb"] + slot],
                device_id=(partner,), device_id_type=pl.DeviceIdType.MESH,
            )

        def exchange(src, dst, partner, s):
            r = pltpu.make_async_remote_copy(
                src_ref=src, dst_ref=dst,
                send_sem=send_sems.at[s], recv_sem=recv_sems.at[s],
                device_id=(partner,), device_id_type=pl.DeviceIdType.MESH,
            )
            r.start()
            return r

        pend = []

        r1 = []
        for c in cfgs:
            h_send = c["bufs"][0]
            p_other = jnp.dot(x_ref[pl.ds(c["other"], H), :],
                              w_ref[:, c["cols"]],
                              preferred_element_type=jnp.float32)
            h_send[...] = p_other.astype(jnp.bfloat16)
            r1.append(exchange(h_send, c["bufs"][1], c["s1"], c["sb"] + 0))
        pend += r1
        keeps = [
            jnp.dot(x_ref[pl.ds(c["keep"], H), :], w_ref[:, c["cols"]],
                    preferred_element_type=jnp.float32)
            for c in cfgs
        ]

        r2 = []
        for c, r, p_keep in zip(cfgs, r1, keeps):
            (_, h_recv, q_send, _, acc) = c["bufs"]
            r.wait_recv()
            acc[...] = p_keep + h_recv[...].astype(jnp.float32)
            q_send[...] = acc[pl.ds(c["qo"], Q), :].astype(jnp.bfloat16)
            r2.append(exchange(q_send, c["bufs"][3], c["s2"], c["sb"] + 1))
        pend += r2

        ag = []
        for c, r in zip(cfgs, r2):
            (_, _, _, q_recv, acc) = c["bufs"]
            r.wait_recv()
            red = acc[pl.ds(c["qw"], Q), :] + q_recv[...].astype(jnp.float32)
            red = jnp.maximum(red, 0.0)
            mine = c["gq"](d)
            out_ref[pl.ds(mine, Q), c["cols"]] = red.astype(jnp.bfloat16)
            s_p1 = rdma(mine, mine, c, 2, p1)
            s_p1.start()
            s_p2 = rdma(mine, mine, c, 3, p2)
            s_p2.start()
            ag.append((s_p1, s_p2))
        pend += [s for pair in ag for s in pair]

        relays = []
        for c in cfgs:
            dp_rows = c["gq"](c["dp"])
            dp_slot = 2 if c["dp"] is p1 else 3
            rcv_dp = rdma(dp_rows, dp_rows, c, dp_slot, c["dp"])
            rcv_dp.wait_recv()
            rs = rdma(dp_rows, dp_rows, c, 4, c["rp"])
            rs.start()
            relays.append(rs)
        pend += relays

        for c in cfgs:
            rp_rows = c["gq"](c["rp"])
            rp_slot = 2 if c["rp"] is p1 else 3
            rdma(rp_rows, rp_rows, c, rp_slot, c["rp"]).wait_recv()
            dg_rows = c["gq"](diag)
            rdma(dg_rows, dg_rows, c, 4, c["rp"]).wait_recv()

        for r in pend:
            r.wait_send()

    lane_scratch = [
        pltpu.VMEM((H, nc), jnp.bfloat16),
        pltpu.VMEM((H, nc), jnp.bfloat16),
        pltpu.VMEM((Q, nc), jnp.bfloat16),
        pltpu.VMEM((Q, nc), jnp.bfloat16),
        pltpu.VMEM((H, nc), jnp.float32),
    ]
    return pl.pallas_call(
        body,
        out_shape=jax.ShapeDtypeStruct((M, n), jnp.bfloat16),
        in_specs=[
            pl.BlockSpec(memory_space=pltpu.VMEM),
            pl.BlockSpec(memory_space=pltpu.VMEM),
        ],
        out_specs=pl.BlockSpec(memory_space=pltpu.VMEM),
        scratch_shapes=lane_scratch * LANES + [
            pltpu.SemaphoreType.DMA((SLOTS * LANES,)),
            pltpu.SemaphoreType.DMA((SLOTS * LANES,)),
        ],
        compiler_params=pltpu.CompilerParams(collective_id=0),
    )(x, w_mat)


# device time: 26780 ns/iter; 1.9159x vs baseline; 1.0482x over previous
import jax
import jax.numpy as jnp
from jax import lax
from jax.experimental import pallas as pl
from jax.experimental.pallas import tpu as pltpu

M = 1024
H = M // 2
Q = M // 4
LANES = 8
BUFS_PER_LANE = 5
SLOTS = 5


def kernel(x, w_mat):
    m, k = x.shape
    _, n = w_mat.shape
    assert m == M
    nc = n // LANES

    def body(x_ref, w_ref, out_ref, *scratch):
        lane_bufs = [
            scratch[i * BUFS_PER_LANE:(i + 1) * BUFS_PER_LANE]
            for i in range(LANES)
        ]
        send_sems, recv_sems = scratch[LANES * BUFS_PER_LANE:]

        d = lax.axis_index("i")
        p1 = d ^ 1
        p2 = 3 - d
        diag = p2 ^ 1
        bit = d >> 1
        low = d & 1

        barrier_sem = pltpu.get_barrier_semaphore()
        for nbr in (p1, p2):
            pl.semaphore_signal(
                barrier_sem, inc=1,
                device_id=(nbr,), device_id_type=pl.DeviceIdType.MESH,
            )
        pl.semaphore_wait(barrier_sem, 2)

        def gq_a(e):
            return ((e & 1) ^ (e >> 1)) * H + (e >> 1) * Q

        def gq_b(e):
            return (e >> 1) * H + (e & 1) * Q

        cfgs = []
        for lane in range(LANES):
            a = lane % 2 == 0
            par = (low ^ bit) if a else bit
            qw = (bit * Q) if a else (low * Q)
            gq = gq_a if a else gq_b
            cfgs.append(dict(
                par=par, qw=qw, qo=Q - qw,
                keep=par * H, other=(1 - par) * H,
                s1=p1 if a else p2,
                s2=p2 if a else p1,
                dp=p1 if a else p2,
                rp=p2 if a else p1,
                gq=gq,
                cols=slice(lane * nc, (lane + 1) * nc),
                sb=SLOTS * lane,
                bufs=lane_bufs[lane],
            ))

        def rdma(src_rows, dst_rows, c, slot, partner):
            return pltpu.make_async_remote_copy(
                src_ref=out_ref.at[pl.ds(src_rows, Q), c["cols"]],
                dst_ref=out_ref.at[pl.ds(dst_rows, Q), c["cols"]],
                send_sem=send_sems.at[c["sb"] + slot],
                recv_sem=recv_sems.at[c["sb"] + slot],
                device_id=(partner,), device_id_type=pl.DeviceIdType.MESH,
            )

        def exchange(src, dst, partner, s):
            r = pltpu.make_async_remote_copy(
                src_ref=src, dst_ref=dst,
                send_sem=send_sems.at[s], recv_sem=recv_sems.at[s],
                device_id=(partner,), device_id_type=pl.DeviceIdType.MESH,
            )
            r.start()
            return r

        pend = []

        r1 = []
        for c in cfgs:
            h_send = c["bufs"][0]
            p_other = jnp.dot(x_ref[pl.ds(c["other"], H), :],
                              w_ref[:, c["cols"]],
                              preferred_element_type=jnp.float32)
            h_send[...] = p_other.astype(jnp.bfloat16)
            r1.append(exchange(h_send, c["bufs"][1], c["s1"], c["sb"] + 0))
        pend += r1
        keeps = [
            jnp.dot(x_ref[pl.ds(c["keep"], H), :], w_ref[:, c["cols"]],
                    preferred_element_type=jnp.float32)
            for c in cfgs
        ]

        r2 = []
        for c, r, p_keep in zip(cfgs, r1, keeps):
            (_, h_recv, q_send, _, acc) = c["bufs"]
            r.wait_recv()
            acc[...] = p_keep + h_recv[...].astype(jnp.float32)
            q_send[...] = acc[pl.ds(c["qo"], Q), :].astype(jnp.bfloat16)
            r2.append(exchange(q_send, c["bufs"][3], c["s2"], c["sb"] + 1))
        pend += r2

        ag = []
        for c, r in zip(cfgs, r2):
            (_, _, _, q_recv, acc) = c["bufs"]
            r.wait_recv()
            red = acc[pl.ds(c["qw"], Q), :] + q_recv[...].astype(jnp.float32)
            red = jnp.maximum(red, 0.0)
            mine = c["gq"](d)
            out_ref[pl.ds(mine, Q), c["cols"]] = red.astype(jnp.bfloat16)
            s_p1 = rdma(mine, mine, c, 2, p1)
            s_p1.start()
            s_p2 = rdma(mine, mine, c, 3, p2)
            s_p2.start()
            ag.append((s_p1, s_p2))
        pend += [s for pair in ag for s in pair]

        relays = []
        for c in cfgs:
            dp_rows = c["gq"](c["dp"])
            dp_slot = 2 if c["dp"] is p1 else 3
            rcv_dp = rdma(dp_rows, dp_rows, c, dp_slot, c["dp"])
            rcv_dp.wait_recv()
            rs = rdma(dp_rows, dp_rows, c, 4, c["rp"])
            rs.start()
            relays.append(rs)
        pend += relays

        for c in cfgs:
            rp_rows = c["gq"](c["rp"])
            rp_slot = 2 if c["rp"] is p1 else 3
            rdma(rp_rows, rp_rows, c, rp_slot, c["rp"]).wait_recv()
            dg_rows = c["gq"](diag)
            rdma(dg_rows, dg_rows, c, 4, c["rp"]).wait_recv()

        for r in pend:
            r.wait_send()

    lane_scratch = [
        pltpu.VMEM((H, nc), jnp.bfloat16),
        pltpu.VMEM((H, nc), jnp.bfloat16),
        pltpu.VMEM((Q, nc), jnp.bfloat16),
        pltpu.VMEM((Q, nc), jnp.bfloat16),
        pltpu.VMEM((H, nc), jnp.float32),
    ]
    return pl.pallas_call(
        body,
        out_shape=jax.ShapeDtypeStruct((M, n), jnp.bfloat16),
        in_specs=[
            pl.BlockSpec(memory_space=pltpu.VMEM),
            pl.BlockSpec(memory_space=pltpu.VMEM),
        ],
        out_specs=pl.BlockSpec(memory_space=pltpu.VMEM),
        scratch_shapes=lane_scratch * LANES + [
            pltpu.SemaphoreType.DMA((SLOTS * LANES,)),
            pltpu.SemaphoreType.DMA((SLOTS * LANES,)),
        ],
        compiler_params=pltpu.CompilerParams(collective_id=0),
    )(x, w_mat)


# device time: 26728 ns/iter; 1.9197x vs baseline; 1.0019x over previous
import jax
import jax.numpy as jnp
from jax import lax
from jax.experimental import pallas as pl
from jax.experimental.pallas import tpu as pltpu

M = 1024
H = M // 2
Q = M // 4
LANES = 8
BUFS_PER_LANE = 4
SLOTS = 5


def kernel(x, w_mat):
    m, k = x.shape
    _, n = w_mat.shape
    assert m == M
    nc = n // LANES

    def body(x_ref, w_ref, out_ref, *scratch):
        lane_bufs = [
            scratch[i * BUFS_PER_LANE:(i + 1) * BUFS_PER_LANE]
            for i in range(LANES)
        ]
        send_sems, recv_sems = scratch[LANES * BUFS_PER_LANE:]

        d = lax.axis_index("i")
        p1 = d ^ 1
        p2 = 3 - d
        diag = p2 ^ 1
        bit = d >> 1
        low = d & 1

        barrier_sem = pltpu.get_barrier_semaphore()
        for nbr in (p1, p2):
            pl.semaphore_signal(
                barrier_sem, inc=1,
                device_id=(nbr,), device_id_type=pl.DeviceIdType.MESH,
            )
        pl.semaphore_wait(barrier_sem, 2)

        def gq_a(e):
            return ((e & 1) ^ (e >> 1)) * H + (e >> 1) * Q

        def gq_b(e):
            return (e >> 1) * H + (e & 1) * Q

        cfgs = []
        for lane in range(LANES):
            a = lane % 2 == 0
            s1 = p1 if a else p2
            s2 = p2 if a else p1
            gq = gq_a if a else gq_b
            s2_of_s1 = (3 - s1) if a else (s1 ^ 1)
            cfgs.append(dict(
                s1=s1, s2=s2,
                dp=p1 if a else p2,
                rp=p2 if a else p1,
                gq=gq, s2_of_s1=s2_of_s1,
                cols=slice(lane * nc, (lane + 1) * nc),
                sb=SLOTS * lane,
                bufs=lane_bufs[lane],
            ))

        def dot_rows(rows, c):
            return jnp.dot(x_ref[pl.ds(rows, Q), :], w_ref[:, c["cols"]],
                           preferred_element_type=jnp.float32)

        def rdma(src_rows, dst_rows, c, slot, partner):
            return pltpu.make_async_remote_copy(
                src_ref=out_ref.at[pl.ds(src_rows, Q), c["cols"]],
                dst_ref=out_ref.at[pl.ds(dst_rows, Q), c["cols"]],
                send_sem=send_sems.at[c["sb"] + slot],
                recv_sem=recv_sems.at[c["sb"] + slot],
                device_id=(partner,), device_id_type=pl.DeviceIdType.MESH,
            )

        def exchange(src, dst, partner, s):
            r = pltpu.make_async_remote_copy(
                src_ref=src, dst_ref=dst,
                send_sem=send_sems.at[s], recv_sem=recv_sems.at[s],
                device_id=(partner,), device_id_type=pl.DeviceIdType.MESH,
            )
            r.start()
            return r

        pend = []

        r1 = []
        for c in cfgs:
            h_send = c["bufs"][0]
            h_send[0:Q, :] = dot_rows(c["gq"](c["s1"]), c).astype(jnp.bfloat16)
            h_send[Q:2 * Q, :] = dot_rows(
                c["gq"](c["s2_of_s1"]), c).astype(jnp.bfloat16)
            r1.append(exchange(h_send, c["bufs"][1], c["s1"], c["sb"] + 0))
        pend += r1

        r2 = []
        for c, r in zip(cfgs, r1):
            (_, h_recv, q_send, _) = c["bufs"]
            r.wait_recv()
            q_send[...] = (dot_rows(c["gq"](c["s2"]), c)
                           + h_recv[Q:2 * Q, :].astype(jnp.float32)
                           ).astype(jnp.bfloat16)
            r2.append(exchange(q_send, c["bufs"][3], c["s2"], c["sb"] + 1))
        pend += r2

        ag = []
        for c, r in zip(cfgs, r2):
            (_, h_recv, _, q_recv) = c["bufs"]
            r.wait_recv()
            red = (dot_rows(c["gq"](d), c)
                   + h_recv[0:Q, :].astype(jnp.float32)
                   + q_recv[...].astype(jnp.float32))
            red = jnp.maximum(red, 0.0)
            mine = c["gq"](d)
            out_ref[pl.ds(mine, Q), c["cols"]] = red.astype(jnp.bfloat16)
            s_p1 = rdma(mine, mine, c, 2, p1)
            s_p1.start()
            s_p2 = rdma(mine, mine, c, 3, p2)
            s_p2.start()
            ag.append((s_p1, s_p2))
        pend += [s for pair in ag for s in pair]

        relays = []
        for c in cfgs:
            dp_rows = c["gq"](c["dp"])
            dp_slot = 2 if c["dp"] is p1 else 3
            rdma(dp_rows, dp_rows, c, dp_slot, c["dp"]).wait_recv()
            rs = rdma(dp_rows, dp_rows, c, 4, c["rp"])
            rs.start()
            relays.append(rs)
        pend += relays

        for c in cfgs:
            rp_rows = c["gq"](c["rp"])
            rp_slot = 2 if c["rp"] is p1 else 3
            rdma(rp_rows, rp_rows, c, rp_slot, c["rp"]).wait_recv()
            dg_rows = c["gq"](diag)
            rdma(dg_rows, dg_rows, c, 4, c["rp"]).wait_recv()

        for r in pend:
            r.wait_send()

    lane_scratch = [
        pltpu.VMEM((2 * Q, nc), jnp.bfloat16),
        pltpu.VMEM((2 * Q, nc), jnp.bfloat16),
        pltpu.VMEM((Q, nc), jnp.bfloat16),
        pltpu.VMEM((Q, nc), jnp.bfloat16),
    ]
    return pl.pallas_call(
        body,
        out_shape=jax.ShapeDtypeStruct((M, n), jnp.bfloat16),
        in_specs=[
            pl.BlockSpec(memory_space=pltpu.VMEM),
            pl.BlockSpec(memory_space=pltpu.VMEM),
        ],
        out_specs=pl.BlockSpec(memory_space=pltpu.VMEM),
        scratch_shapes=lane_scratch * LANES + [
            pltpu.SemaphoreType.DMA((SLOTS * LANES,)),
            pltpu.SemaphoreType.DMA((SLOTS * LANES,)),
        ],
        compiler_params=pltpu.CompilerParams(collective_id=0),
    )(x, w_mat)


# device time: 26092 ns/iter; 1.9665x vs baseline; 1.0244x over previous
import jax
import jax.numpy as jnp
from jax import lax
from jax.experimental import pallas as pl
from jax.experimental.pallas import tpu as pltpu

M = 1024
H = M // 2
Q = M // 4
LANES = 8
BUFS_PER_LANE = 4
SLOTS = 6


def kernel(x, w_mat):
    m, k = x.shape
    _, n = w_mat.shape
    assert m == M
    nc = n // LANES

    def body(x_ref, w_ref, out_ref, *scratch):
        lane_bufs = [
            scratch[i * BUFS_PER_LANE:(i + 1) * BUFS_PER_LANE]
            for i in range(LANES)
        ]
        send_sems, recv_sems = scratch[LANES * BUFS_PER_LANE:]

        d = lax.axis_index("i")
        p1 = d ^ 1
        p2 = 3 - d
        diag = p2 ^ 1
        bit = d >> 1
        low = d & 1

        barrier_sem = pltpu.get_barrier_semaphore()
        for nbr in (p1, p2):
            pl.semaphore_signal(
                barrier_sem, inc=1,
                device_id=(nbr,), device_id_type=pl.DeviceIdType.MESH,
            )
        pl.semaphore_wait(barrier_sem, 2)

        def gq_a(e):
            return ((e & 1) ^ (e >> 1)) * H + (e >> 1) * Q

        def gq_b(e):
            return (e >> 1) * H + (e & 1) * Q

        cfgs = []
        for lane in range(LANES):
            a = lane % 2 == 0
            s1 = p1 if a else p2
            s2 = p2 if a else p1
            gq = gq_a if a else gq_b
            s2_of_s1 = (3 - s1) if a else (s1 ^ 1)
            cfgs.append(dict(
                s1=s1, s2=s2,
                dp=p1 if a else p2,
                rp=p2 if a else p1,
                gq=gq, s2_of_s1=s2_of_s1,
                cols=slice(lane * nc, (lane + 1) * nc),
                sb=SLOTS * lane,
                bufs=lane_bufs[lane],
            ))

        def dot_rows(rows, c):
            return jnp.dot(x_ref[pl.ds(rows, Q), :], w_ref[:, c["cols"]],
                           preferred_element_type=jnp.float32)

        def rdma(src_rows, dst_rows, c, slot, partner):
            return pltpu.make_async_remote_copy(
                src_ref=out_ref.at[pl.ds(src_rows, Q), c["cols"]],
                dst_ref=out_ref.at[pl.ds(dst_rows, Q), c["cols"]],
                send_sem=send_sems.at[c["sb"] + slot],
                recv_sem=recv_sems.at[c["sb"] + slot],
                device_id=(partner,), device_id_type=pl.DeviceIdType.MESH,
            )

        def exchange(src, dst, partner, s):
            r = pltpu.make_async_remote_copy(
                src_ref=src, dst_ref=dst,
                send_sem=send_sems.at[s], recv_sem=recv_sems.at[s],
                device_id=(partner,), device_id_type=pl.DeviceIdType.MESH,
            )
            r.start()
            return r

        pend = []

        r1b = []
        for c in cfgs:
            h_send = c["bufs"][0]
            h_send[Q:2 * Q, :] = dot_rows(
                c["gq"](c["s2_of_s1"]), c).astype(jnp.bfloat16)
            r1b.append(exchange(h_send.at[pl.ds(Q, Q), :],
                                c["bufs"][1].at[pl.ds(Q, Q), :],
                                c["s1"], c["sb"] + 0))
        pend += r1b

        r1a = []
        for c in cfgs:
            h_send = c["bufs"][0]
            h_send[0:Q, :] = dot_rows(c["gq"](c["s1"]), c).astype(jnp.bfloat16)
            r1a.append(exchange(h_send.at[pl.ds(0, Q), :],
                                c["bufs"][1].at[pl.ds(0, Q), :],
                                c["s1"], c["sb"] + 1))
        pend += r1a

        r2 = []
        for c, r in zip(cfgs, r1b):
            (_, h_recv, q_send, _) = c["bufs"]
            r.wait_recv()
            q_send[...] = (dot_rows(c["gq"](c["s2"]), c)
                           + h_recv[Q:2 * Q, :].astype(jnp.float32)
                           ).astype(jnp.bfloat16)
            r2.append(exchange(q_send, c["bufs"][3], c["s2"], c["sb"] + 2))
        pend += r2

        ag = []
        for c, ra, r in zip(cfgs, r1a, r2):
            (_, h_recv, _, q_recv) = c["bufs"]
            ra.wait_recv()
            r.wait_recv()
            red = (dot_rows(c["gq"](d), c)
                   + h_recv[0:Q, :].astype(jnp.float32)
                   + q_recv[...].astype(jnp.float32))
            red = jnp.maximum(red, 0.0)
            mine = c["gq"](d)
            out_ref[pl.ds(mine, Q), c["cols"]] = red.astype(jnp.bfloat16)
            s_p1 = rdma(mine, mine, c, 3, p1)
            s_p1.start()
            s_p2 = rdma(mine, mine, c, 4, p2)
            s_p2.start()
            ag.append((s_p1, s_p2))
        pend += [s for pair in ag for s in pair]

        relays = []
        for c in cfgs:
            dp_rows = c["gq"](c["dp"])
            dp_slot = 3 if c["dp"] is p1 else 4
            rdma(dp_rows, dp_rows, c, dp_slot, c["dp"]).wait_recv()
            rs = rdma(dp_rows, dp_rows, c, 5, c["rp"])
            rs.start()
            relays.append(rs)
        pend += relays

        for c in cfgs:
            rp_rows = c["gq"](c["rp"])
            rp_slot = 3 if c["rp"] is p1 else 4
            rdma(rp_rows, rp_rows, c, rp_slot, c["rp"]).wait_recv()
            dg_rows = c["gq"](diag)
            rdma(dg_rows, dg_rows, c, 5, c["rp"]).wait_recv()

        for r in pend:
            r.wait_send()

    lane_scratch = [
        pltpu.VMEM((2 * Q, nc), jnp.bfloat16),
        pltpu.VMEM((2 * Q, nc), jnp.bfloat16),
        pltpu.VMEM((Q, nc), jnp.bfloat16),
        pltpu.VMEM((Q, nc), jnp.bfloat16),
    ]
    return pl.pallas_call(
        body,
        out_shape=jax.ShapeDtypeStruct((M, n), jnp.bfloat16),
        in_specs=[
            pl.BlockSpec(memory_space=pltpu.VMEM),
            pl.BlockSpec(memory_space=pltpu.VMEM),
        ],
        out_specs=pl.BlockSpec(memory_space=pltpu.VMEM),
        scratch_shapes=lane_scratch * LANES + [
            pltpu.SemaphoreType.DMA((SLOTS * LANES,)),
            pltpu.SemaphoreType.DMA((SLOTS * LANES,)),
        ],
        compiler_params=pltpu.CompilerParams(collective_id=0),
    )(x, w_mat)
